# baseline (speedup 1.0000x reference)
import functools

import jax
import jax.numpy as jnp
from jax import lax
from jax.experimental import pallas as pl
from jax.experimental.pallas import tpu as pltpu
from jax.experimental.pallas import tpu_sc as plsc

B = 16384
D = 128
NC, NS = 2, 16
NW = NC * NS
CH = 128

K = 4
BK = B // K
B2K = 2 * BK
BPW = B2K // NW
NCHUNK = BPW // CH


def _gather_rows(table, idx2d):
    mesh = plsc.VectorSubcoreMesh(
        core_axis_name="c", subcore_axis_name="s",
        num_cores=NC, num_subcores=NS)

    @functools.partial(
        pl.kernel,
        out_type=jax.ShapeDtypeStruct((B2K, D), jnp.float32),
        mesh=mesh,
        scratch_types=[
            pltpu.VMEM((NCHUNK, CH), jnp.int32),
            pltpu.VMEM((CH, D), jnp.float32),
            pltpu.VMEM((CH, D), jnp.float32),
            pltpu.SemaphoreType.DMA,
            pltpu.SemaphoreType.DMA,
        ],
    )
    def gather_kernel(table_hbm, idx_hbm, out_hbm, idx_v, buf0, buf1, sem0, sem1):
        wid = lax.axis_index("s") * NC + lax.axis_index("c")
        base = wid * BPW
        pltpu.sync_copy(idx_hbm.at[pl.ds(wid * NCHUNK, NCHUNK)], idx_v)
        bufs = (buf0, buf1)
        sems = (sem0, sem1)
        copies = [None] * NCHUNK
        copies[0] = pltpu.async_copy(table_hbm.at[idx_v.at[0]], bufs[0], sems[0])
        for j in range(1, NCHUNK):
            copies[j] = pltpu.async_copy(
                table_hbm.at[idx_v.at[j]], bufs[j % 2], sems[j % 2])
            copies[j - 1].wait()
            pltpu.sync_copy(bufs[(j - 1) % 2],
                            out_hbm.at[pl.ds(base + (j - 1) * CH, CH)])
        copies[NCHUNK - 1].wait()
        pltpu.sync_copy(bufs[(NCHUNK - 1) % 2],
                        out_hbm.at[pl.ds(base + (NCHUNK - 1) * CH, CH)])

    return gather_kernel(table, idx2d)


def _mlp_body(eng_ref, grk_ref, w1a_ref, w1b_ref, b1_ref, w2_ref, b2_ref,
              w3_ref, b3_ref, out_ref):
    h = eng_ref[...] @ w1a_ref[...] + grk_ref[...] @ w1b_ref[...] + b1_ref[...]
    h = jnp.maximum(h, 0.0)
    h = jnp.maximum(h @ w2_ref[...] + b2_ref[...], 0.0)
    z = jnp.sum(h * w3_ref[...], axis=1, keepdims=True) + b3_ref[...]
    out_ref[...] = 1.0 / (1.0 + jnp.exp(-z))


def _mlp(emb, W1a, W1b, b1, W2, b2, W3t, b3):
    BLK = 1024
    nblk = BK // BLK
    full = lambda shape: pl.BlockSpec(shape, lambda i: (0, 0))
    return pl.pallas_call(
        _mlp_body,
        grid=(nblk,),
        in_specs=[
            pl.BlockSpec((BLK, D), lambda i: (i, 0)),
            pl.BlockSpec((BLK, D), lambda i: (i + nblk, 0)),
            full((D, D)),
            full((D, D)),
            full((1, D)),
            full((D, 64)),
            full((1, 64)),
            full((1, 64)),
            full((1, 1)),
        ],
        out_specs=pl.BlockSpec((BLK, 1), lambda i: (i, 0)),
        out_shape=jax.ShapeDtypeStruct((BK, 1), jnp.float32),
    )(emb, emb, W1a, W1b, b1, W2, b2, W3t, b3)


def kernel(eng_ids, grk_ids, table, W1, b1, W2, b2, W3, b3):
    eng = eng_ids.astype(jnp.int32)
    grk = grk_ids.astype(jnp.int32)
    W1a, W1b = W1[:D], W1[D:]
    b1r = b1.reshape(1, D)
    b2r = b2.reshape(1, 64)
    W3t = W3.reshape(1, 64)
    b3r = b3.reshape(1, 1)
    outs = []
    for k in range(K):
        idx = jnp.concatenate(
            [lax.dynamic_slice(eng, (k * BK,), (BK,)),
             lax.dynamic_slice(grk, (k * BK,), (BK,))])
        emb = _gather_rows(table, idx.reshape(NW * NCHUNK, CH))
        outs.append(_mlp(emb, W1a, W1b, b1r, W2, b2r, W3t, b3r))
    return jnp.concatenate(outs)


# speedup vs baseline: 1.0902x; 1.0902x over previous
import functools

import jax
import jax.numpy as jnp
from jax import lax
from jax.experimental import pallas as pl
from jax.experimental.pallas import tpu as pltpu
from jax.experimental.pallas import tpu_sc as plsc

B = 16384
D = 128
NC, NS = 2, 16
NW = NC * NS
CH = 128

K = 2
BK = B // K
B2K = 2 * BK
BPW = B2K // NW
NCHUNK = BPW // CH


def _gather_rows(table, idx2d):
    mesh = plsc.VectorSubcoreMesh(
        core_axis_name="c", subcore_axis_name="s",
        num_cores=NC, num_subcores=NS)

    @functools.partial(
        pl.kernel,
        out_type=jax.ShapeDtypeStruct((B2K, D), jnp.float32),
        mesh=mesh,
        scratch_types=[
            pltpu.VMEM((NCHUNK, CH), jnp.int32),
            pltpu.VMEM((CH, D), jnp.float32),
            pltpu.VMEM((CH, D), jnp.float32),
            pltpu.SemaphoreType.DMA,
            pltpu.SemaphoreType.DMA,
        ],
    )
    def gather_kernel(table_hbm, idx_hbm, out_hbm, idx_v, buf0, buf1, sem0, sem1):
        wid = lax.axis_index("s") * NC + lax.axis_index("c")
        base = wid * BPW
        pltpu.sync_copy(idx_hbm.at[pl.ds(wid * NCHUNK, NCHUNK)], idx_v)
        bufs = (buf0, buf1)
        sems = (sem0, sem1)
        copies = [None] * NCHUNK
        copies[0] = pltpu.async_copy(table_hbm.at[idx_v.at[0]], bufs[0], sems[0])
        for j in range(1, NCHUNK):
            copies[j] = pltpu.async_copy(
                table_hbm.at[idx_v.at[j]], bufs[j % 2], sems[j % 2])
            copies[j - 1].wait()
            pltpu.sync_copy(bufs[(j - 1) % 2],
                            out_hbm.at[pl.ds(base + (j - 1) * CH, CH)])
        copies[NCHUNK - 1].wait()
        pltpu.sync_copy(bufs[(NCHUNK - 1) % 2],
                        out_hbm.at[pl.ds(base + (NCHUNK - 1) * CH, CH)])

    return gather_kernel(table, idx2d)


def _mlp_body(eng_ref, grk_ref, w1a_ref, w1b_ref, b1_ref, w2_ref, b2_ref,
              w3_ref, b3_ref, out_ref):
    h = eng_ref[...] @ w1a_ref[...] + grk_ref[...] @ w1b_ref[...] + b1_ref[...]
    h = jnp.maximum(h, 0.0)
    h = jnp.maximum(h @ w2_ref[...] + b2_ref[...], 0.0)
    z = jnp.sum(h * w3_ref[...], axis=1, keepdims=True) + b3_ref[...]
    out_ref[...] = 1.0 / (1.0 + jnp.exp(-z))


def _mlp(emb, W1a, W1b, b1, W2, b2, W3t, b3):
    BLK = 1024
    nblk = BK // BLK
    full = lambda shape: pl.BlockSpec(shape, lambda i: (0, 0))
    return pl.pallas_call(
        _mlp_body,
        grid=(nblk,),
        in_specs=[
            pl.BlockSpec((BLK, D), lambda i: (i, 0)),
            pl.BlockSpec((BLK, D), lambda i: (i + nblk, 0)),
            full((D, D)),
            full((D, D)),
            full((1, D)),
            full((D, 64)),
            full((1, 64)),
            full((1, 64)),
            full((1, 1)),
        ],
        out_specs=pl.BlockSpec((BLK, 1), lambda i: (i, 0)),
        out_shape=jax.ShapeDtypeStruct((BK, 1), jnp.float32),
    )(emb, emb, W1a, W1b, b1, W2, b2, W3t, b3)


def kernel(eng_ids, grk_ids, table, W1, b1, W2, b2, W3, b3):
    eng = eng_ids.astype(jnp.int32)
    grk = grk_ids.astype(jnp.int32)
    W1a, W1b = W1[:D], W1[D:]
    b1r = b1.reshape(1, D)
    b2r = b2.reshape(1, 64)
    W3t = W3.reshape(1, 64)
    b3r = b3.reshape(1, 1)
    outs = []
    for k in range(K):
        idx = jnp.concatenate(
            [lax.dynamic_slice(eng, (k * BK,), (BK,)),
             lax.dynamic_slice(grk, (k * BK,), (BK,))])
        emb = _gather_rows(table, idx.reshape(NW * NCHUNK, CH))
        outs.append(_mlp(emb, W1a, W1b, b1r, W2, b2r, W3t, b3r))
    return jnp.concatenate(outs)


# speedup vs baseline: 1.1392x; 1.0449x over previous
import functools

import jax
import jax.numpy as jnp
from jax import lax
from jax.experimental import pallas as pl
from jax.experimental.pallas import tpu as pltpu
from jax.experimental.pallas import tpu_sc as plsc

B = 16384
D = 128
NC, NS = 2, 16
NW = NC * NS
CH = 128

K = 2
BK = B // K
ROWS = BK // CH // NW
NCHUNK = 2 * ROWS


def _gather_rows(table, eng2d, grk2d, k):
    mesh = plsc.VectorSubcoreMesh(
        core_axis_name="c", subcore_axis_name="s",
        num_cores=NC, num_subcores=NS)

    @functools.partial(
        pl.kernel,
        out_type=jax.ShapeDtypeStruct((2 * BK, D), jnp.float32),
        mesh=mesh,
        scratch_types=[
            pltpu.VMEM((NCHUNK, CH), jnp.int32),
            pltpu.VMEM((CH, D), jnp.float32),
            pltpu.VMEM((CH, D), jnp.float32),
            pltpu.SemaphoreType.DMA,
            pltpu.SemaphoreType.DMA,
        ],
    )
    def gather_kernel(table_hbm, eng_hbm, grk_hbm, out_hbm,
                      idx_v, buf0, buf1, sem0, sem1):
        wid = lax.axis_index("s") * NC + lax.axis_index("c")
        row0 = k * (BK // CH) + wid * ROWS
        pltpu.sync_copy(eng_hbm.at[pl.ds(row0, ROWS)], idx_v.at[pl.ds(0, ROWS)])
        pltpu.sync_copy(grk_hbm.at[pl.ds(row0, ROWS)],
                        idx_v.at[pl.ds(ROWS, ROWS)])
        dst = [wid * ROWS * CH + j * CH for j in range(ROWS)]
        dst += [BK + wid * ROWS * CH + j * CH for j in range(ROWS)]
        bufs = (buf0, buf1)
        sems = (sem0, sem1)
        copies = [None] * NCHUNK
        copies[0] = pltpu.async_copy(table_hbm.at[idx_v.at[0]], bufs[0], sems[0])
        for j in range(1, NCHUNK):
            copies[j] = pltpu.async_copy(
                table_hbm.at[idx_v.at[j]], bufs[j % 2], sems[j % 2])
            copies[j - 1].wait()
            pltpu.sync_copy(bufs[(j - 1) % 2],
                            out_hbm.at[pl.ds(dst[j - 1], CH)])
        copies[NCHUNK - 1].wait()
        pltpu.sync_copy(bufs[(NCHUNK - 1) % 2],
                        out_hbm.at[pl.ds(dst[NCHUNK - 1], CH)])

    return gather_kernel(table, eng2d, grk2d)


def _mlp_body(eng_ref, grk_ref, w1a_ref, w1b_ref, b1_ref, w2_ref, b2_ref,
              w3_ref, b3_ref, out_ref):
    h = eng_ref[...] @ w1a_ref[...] + grk_ref[...] @ w1b_ref[...] + b1_ref[...]
    h = jnp.maximum(h, 0.0)
    h = jnp.maximum(h @ w2_ref[...] + b2_ref[...], 0.0)
    z = jnp.sum(h * w3_ref[...], axis=1, keepdims=True) + b3_ref[...]
    out_ref[...] = 1.0 / (1.0 + jnp.exp(-z))


def _mlp_alias_body(big_ref, eng_ref, grk_ref, w1a_ref, w1b_ref, b1_ref,
                    w2_ref, b2_ref, w3_ref, b3_ref, out_ref):
    del big_ref
    _mlp_body(eng_ref, grk_ref, w1a_ref, w1b_ref, b1_ref, w2_ref, b2_ref,
              w3_ref, b3_ref, out_ref)


BLK = 2048
NBLK = BK // BLK


def _mlp(emb, k, big, W1a, W1b, b1, W2, b2, W3t, b3):
    full = lambda shape: pl.BlockSpec(shape, lambda i: (0, 0))
    in_specs = [
        pl.BlockSpec((BLK, D), lambda i: (i, 0)),
        pl.BlockSpec((BLK, D), lambda i: (i + NBLK, 0)),
        full((D, D)),
        full((D, D)),
        full((1, D)),
        full((D, 64)),
        full((1, 64)),
        full((1, 64)),
        full((1, 1)),
    ]
    args = (emb, emb, W1a, W1b, b1, W2, b2, W3t, b3)
    body = _mlp_body
    kwargs = {}
    if big is not None:
        in_specs = [pl.BlockSpec(memory_space=pl.ANY)] + in_specs
        args = (big,) + args
        body = _mlp_alias_body
        kwargs = {"input_output_aliases": {0: 0}}
    return pl.pallas_call(
        body,
        grid=(NBLK,),
        in_specs=in_specs,
        out_specs=pl.BlockSpec((BLK, 1), lambda i, k=k: (i + k * NBLK, 0)),
        out_shape=jax.ShapeDtypeStruct((B, 1), jnp.float32),
        **kwargs,
    )(*args)


def kernel(eng_ids, grk_ids, table, W1, b1, W2, b2, W3, b3):
    eng2d = eng_ids.astype(jnp.int32).reshape(B // CH, CH)
    grk2d = grk_ids.astype(jnp.int32).reshape(B // CH, CH)
    W1a, W1b = W1[:D], W1[D:]
    b1r = b1.reshape(1, D)
    b2r = b2.reshape(1, 64)
    W3t = W3.reshape(1, 64)
    b3r = b3.reshape(1, 1)
    out = None
    for k in range(K):
        emb = _gather_rows(table, eng2d, grk2d, k)
        out = _mlp(emb, k, out, W1a, W1b, b1r, W2, b2r, W3t, b3r)
    return out


# speedup vs baseline: 1.3066x; 1.1470x over previous
import functools

import jax
import jax.numpy as jnp
from jax import lax
from jax.experimental import pallas as pl
from jax.experimental.pallas import tpu as pltpu
from jax.experimental.pallas import tpu_sc as plsc

B = 16384
D = 128
NC, NS = 2, 16
NW = NC * NS
CH = 128

K = 2
BK = B // K
ROWS = BK // CH // NW
NCHUNK = 2 * ROWS


def _gather_rows(table, eng2d, grk2d, k):
    mesh = plsc.VectorSubcoreMesh(
        core_axis_name="c", subcore_axis_name="s",
        num_cores=NC, num_subcores=NS)

    @functools.partial(
        pl.kernel,
        out_type=jax.ShapeDtypeStruct((2 * BK, D), jnp.float32),
        mesh=mesh,
        scratch_types=[
            pltpu.VMEM((NCHUNK, CH), jnp.int32),
            pltpu.VMEM((CH, D), jnp.float32),
            pltpu.VMEM((CH, D), jnp.float32),
            pltpu.SemaphoreType.DMA,
            pltpu.SemaphoreType.DMA,
        ],
    )
    def gather_kernel(table_hbm, eng_hbm, grk_hbm, out_hbm,
                      idx_v, buf0, buf1, sem0, sem1):
        wid = lax.axis_index("s") * NC + lax.axis_index("c")
        row0 = k * (BK // CH) + wid * ROWS
        pltpu.sync_copy(eng_hbm.at[pl.ds(row0, ROWS)], idx_v.at[pl.ds(0, ROWS)])
        pltpu.sync_copy(grk_hbm.at[pl.ds(row0, ROWS)],
                        idx_v.at[pl.ds(ROWS, ROWS)])
        dst = [wid * ROWS * CH + j * CH for j in range(ROWS)]
        dst += [BK + wid * ROWS * CH + j * CH for j in range(ROWS)]
        bufs = (buf0, buf1)
        sems = (sem0, sem1)
        copies = [None] * NCHUNK
        copies[0] = pltpu.async_copy(table_hbm.at[idx_v.at[0]], bufs[0], sems[0])
        for j in range(1, NCHUNK):
            copies[j] = pltpu.async_copy(
                table_hbm.at[idx_v.at[j]], bufs[j % 2], sems[j % 2])
            copies[j - 1].wait()
            pltpu.sync_copy(bufs[(j - 1) % 2],
                            out_hbm.at[pl.ds(dst[j - 1], CH)])
        copies[NCHUNK - 1].wait()
        pltpu.sync_copy(bufs[(NCHUNK - 1) % 2],
                        out_hbm.at[pl.ds(dst[NCHUNK - 1], CH)])

    return gather_kernel(table, eng2d, grk2d)


def _mlp_body(eng_ref, grk_ref, w1a_ref, w1b_ref, b1_ref, w2_ref, b2_ref,
              w3_ref, b3_ref, out_ref):
    h = eng_ref[...] @ w1a_ref[...] + grk_ref[...] @ w1b_ref[...] + b1_ref[...]
    h = jnp.maximum(h, 0.0)
    h = jnp.maximum(h @ w2_ref[...] + b2_ref[...], 0.0)
    zt = lax.dot_general(w3_ref[...], h, (((1,), (1,)), ((), ()))) + b3_ref[...]
    out_ref[...] = 1.0 / (1.0 + jnp.exp(-zt))


def _mlp_alias_body(big_ref, eng_ref, grk_ref, w1a_ref, w1b_ref, b1_ref,
                    w2_ref, b2_ref, w3_ref, b3_ref, out_ref):
    del big_ref
    _mlp_body(eng_ref, grk_ref, w1a_ref, w1b_ref, b1_ref, w2_ref, b2_ref,
              w3_ref, b3_ref, out_ref)


BLK = 2048
NBLK = BK // BLK


def _mlp(emb, k, big, W1a, W1b, b1, W2, b2, W3t, b3):
    full = lambda shape: pl.BlockSpec(shape, lambda i: (0, 0))
    in_specs = [
        pl.BlockSpec((BLK, D), lambda i: (i, 0)),
        pl.BlockSpec((BLK, D), lambda i: (i + NBLK, 0)),
        full((D, D)),
        full((D, D)),
        full((1, D)),
        full((D, 64)),
        full((1, 64)),
        full((1, 64)),
        full((1, 1)),
    ]
    args = (emb, emb, W1a, W1b, b1, W2, b2, W3t, b3)
    body = _mlp_body
    kwargs = {}
    if big is not None:
        in_specs = [pl.BlockSpec(memory_space=pl.ANY)] + in_specs
        args = (big,) + args
        body = _mlp_alias_body
        kwargs = {"input_output_aliases": {0: 0}}
    return pl.pallas_call(
        body,
        grid=(NBLK,),
        in_specs=in_specs,
        out_specs=pl.BlockSpec((1, BLK), lambda i, k=k: (0, i + k * NBLK)),
        out_shape=jax.ShapeDtypeStruct((1, B), jnp.float32),
        **kwargs,
    )(*args)


def kernel(eng_ids, grk_ids, table, W1, b1, W2, b2, W3, b3):
    eng2d = eng_ids.astype(jnp.int32).reshape(B // CH, CH)
    grk2d = grk_ids.astype(jnp.int32).reshape(B // CH, CH)
    W1a, W1b = W1[:D], W1[D:]
    b1r = b1.reshape(1, D)
    b2r = b2.reshape(1, 64)
    W3t = W3.reshape(1, 64)
    b3r = b3.reshape(1, 1)
    out = None
    for k in range(K):
        emb = _gather_rows(table, eng2d, grk2d, k)
        out = _mlp(emb, k, out, W1a, W1b, b1r, W2, b2r, W3t, b3r)
    return out.reshape(B, 1)


# speedup vs baseline: 1.3132x; 1.0051x over previous
import functools

import jax
import jax.numpy as jnp
from jax import lax
from jax.experimental import pallas as pl
from jax.experimental.pallas import tpu as pltpu
from jax.experimental.pallas import tpu_sc as plsc

B = 16384
D = 128
NC, NS = 2, 16
NW = NC * NS
CH = 128

K = 2
BK = B // K
ROWS = BK // CH // NW
NCHUNK = 2 * ROWS


def _gather_rows(table, eng2d, grk2d, k):
    mesh = plsc.VectorSubcoreMesh(
        core_axis_name="c", subcore_axis_name="s",
        num_cores=NC, num_subcores=NS)

    @functools.partial(
        pl.kernel,
        out_type=jax.ShapeDtypeStruct((2 * BK, D), jnp.float32),
        mesh=mesh,
        scratch_types=[
            pltpu.VMEM((NCHUNK, CH), jnp.int32),
            pltpu.VMEM((CH, D), jnp.float32),
            pltpu.VMEM((CH, D), jnp.float32),
            pltpu.SemaphoreType.DMA,
            pltpu.SemaphoreType.DMA,
        ],
    )
    def gather_kernel(table_hbm, eng_hbm, grk_hbm, out_hbm,
                      idx_v, buf0, buf1, sem0, sem1):
        wid = lax.axis_index("s") * NC + lax.axis_index("c")
        row0 = k * (BK // CH) + wid * ROWS
        pltpu.sync_copy(eng_hbm.at[pl.ds(row0, ROWS)], idx_v.at[pl.ds(0, ROWS)])
        pltpu.sync_copy(grk_hbm.at[pl.ds(row0, ROWS)],
                        idx_v.at[pl.ds(ROWS, ROWS)])
        dst = [wid * ROWS * CH + j * CH for j in range(ROWS)]
        dst += [BK + wid * ROWS * CH + j * CH for j in range(ROWS)]
        bufs = (buf0, buf1)
        sems = (sem0, sem1)
        copies = [None] * NCHUNK
        copies[0] = pltpu.async_copy(table_hbm.at[idx_v.at[0]], bufs[0], sems[0])
        for j in range(1, NCHUNK):
            copies[j] = pltpu.async_copy(
                table_hbm.at[idx_v.at[j]], bufs[j % 2], sems[j % 2])
            copies[j - 1].wait()
            pltpu.sync_copy(bufs[(j - 1) % 2],
                            out_hbm.at[pl.ds(dst[j - 1], CH)])
        copies[NCHUNK - 1].wait()
        pltpu.sync_copy(bufs[(NCHUNK - 1) % 2],
                        out_hbm.at[pl.ds(dst[NCHUNK - 1], CH)])

    return gather_kernel(table, eng2d, grk2d)


def _mlp_body(eng_ref, grk_ref, w1a_ref, w1b_ref, b1_ref, w2_ref, b2_ref,
              w3_ref, b3_ref, out_ref):
    h = eng_ref[...] @ w1a_ref[...] + grk_ref[...] @ w1b_ref[...] + b1_ref[...]
    h = jnp.maximum(h, 0.0)
    h = jnp.maximum(h @ w2_ref[...] + b2_ref[...], 0.0)
    zt = lax.dot_general(w3_ref[...], h, (((1,), (1,)), ((), ()))) + b3_ref[...]
    out_ref[...] = 1.0 / (1.0 + jnp.exp(-zt))


def _mlp_alias_body(big_ref, eng_ref, grk_ref, w1a_ref, w1b_ref, b1_ref,
                    w2_ref, b2_ref, w3_ref, b3_ref, out_ref):
    del big_ref
    _mlp_body(eng_ref, grk_ref, w1a_ref, w1b_ref, b1_ref, w2_ref, b2_ref,
              w3_ref, b3_ref, out_ref)


BLK = 4096
NBLK = BK // BLK


def _mlp(emb, k, big, W1a, W1b, b1, W2, b2, W3t, b3):
    full = lambda shape: pl.BlockSpec(shape, lambda i: (0, 0))
    in_specs = [
        pl.BlockSpec((BLK, D), lambda i: (i, 0)),
        pl.BlockSpec((BLK, D), lambda i: (i + NBLK, 0)),
        full((D, D)),
        full((D, D)),
        full((1, D)),
        full((D, 64)),
        full((1, 64)),
        full((1, 64)),
        full((1, 1)),
    ]
    args = (emb, emb, W1a, W1b, b1, W2, b2, W3t, b3)
    body = _mlp_body
    kwargs = {}
    if big is not None:
        in_specs = [pl.BlockSpec(memory_space=pl.ANY)] + in_specs
        args = (big,) + args
        body = _mlp_alias_body
        kwargs = {"input_output_aliases": {0: 0}}
    return pl.pallas_call(
        body,
        grid=(NBLK,),
        in_specs=in_specs,
        out_specs=pl.BlockSpec((1, BLK), lambda i, k=k: (0, i + k * NBLK)),
        out_shape=jax.ShapeDtypeStruct((1, B), jnp.float32),
        **kwargs,
    )(*args)


def kernel(eng_ids, grk_ids, table, W1, b1, W2, b2, W3, b3):
    eng2d = eng_ids.astype(jnp.int32).reshape(B // CH, CH)
    grk2d = grk_ids.astype(jnp.int32).reshape(B // CH, CH)
    W1a, W1b = W1[:D], W1[D:]
    b1r = b1.reshape(1, D)
    b2r = b2.reshape(1, 64)
    W3t = W3.reshape(1, 64)
    b3r = b3.reshape(1, 1)
    out = None
    for k in range(K):
        emb = _gather_rows(table, eng2d, grk2d, k)
        out = _mlp(emb, k, out, W1a, W1b, b1r, W2, b2r, W3t, b3r)
    return out.reshape(B, 1)


# speedup vs baseline: 1.3688x; 1.0424x over previous
import functools

import jax
import jax.numpy as jnp
from jax import lax
from jax.experimental import pallas as pl
from jax.experimental.pallas import tpu as pltpu
from jax.experimental.pallas import tpu_sc as plsc

B = 16384
D = 128
NC, NS = 2, 16
NW = NC * NS
CH = 128

K = 2
BK = B // K
ROWS = BK // CH // NW
NCHUNK = 2 * ROWS


def _gather_rows(table, eng2d, grk2d, k):
    mesh = plsc.VectorSubcoreMesh(
        core_axis_name="c", subcore_axis_name="s",
        num_cores=NC, num_subcores=NS)

    @functools.partial(
        pl.kernel,
        out_type=jax.ShapeDtypeStruct((2 * BK, D), jnp.float32),
        mesh=mesh,
        scratch_types=(
            [pltpu.VMEM((NCHUNK, CH), jnp.int32)]
            + [pltpu.VMEM((CH, D), jnp.float32)] * NCHUNK
            + [pltpu.SemaphoreType.DMA] * (2 * NCHUNK)
        ),
    )
    def gather_kernel(table_hbm, eng_hbm, grk_hbm, out_hbm, idx_v, *rest):
        bufs = rest[:NCHUNK]
        gsems = rest[NCHUNK:2 * NCHUNK]
        ssems = rest[2 * NCHUNK:]
        wid = lax.axis_index("s") * NC + lax.axis_index("c")
        row0 = k * (BK // CH) + wid * ROWS
        pltpu.sync_copy(eng_hbm.at[pl.ds(row0, ROWS)], idx_v.at[pl.ds(0, ROWS)])
        pltpu.sync_copy(grk_hbm.at[pl.ds(row0, ROWS)],
                        idx_v.at[pl.ds(ROWS, ROWS)])
        dst = [wid * ROWS * CH + j * CH for j in range(ROWS)]
        dst += [BK + wid * ROWS * CH + j * CH for j in range(ROWS)]
        gathers = [
            pltpu.async_copy(table_hbm.at[idx_v.at[j]], bufs[j], gsems[j])
            for j in range(NCHUNK)
        ]
        stores = [None] * NCHUNK
        for j in range(NCHUNK):
            gathers[j].wait()
            stores[j] = pltpu.async_copy(
                bufs[j], out_hbm.at[pl.ds(dst[j], CH)], ssems[j])
        for j in range(NCHUNK):
            stores[j].wait()

    return gather_kernel(table, eng2d, grk2d)


def _mlp_body(eng_ref, grk_ref, w1a_ref, w1b_ref, b1_ref, w2_ref, b2_ref,
              w3_ref, b3_ref, out_ref):
    h = eng_ref[...] @ w1a_ref[...] + grk_ref[...] @ w1b_ref[...] + b1_ref[...]
    h = jnp.maximum(h, 0.0)
    h = jnp.maximum(h @ w2_ref[...] + b2_ref[...], 0.0)
    zt = lax.dot_general(w3_ref[...], h, (((1,), (1,)), ((), ()))) + b3_ref[...]
    out_ref[...] = 1.0 / (1.0 + jnp.exp(-zt))


def _mlp_alias_body(big_ref, eng_ref, grk_ref, w1a_ref, w1b_ref, b1_ref,
                    w2_ref, b2_ref, w3_ref, b3_ref, out_ref):
    del big_ref
    _mlp_body(eng_ref, grk_ref, w1a_ref, w1b_ref, b1_ref, w2_ref, b2_ref,
              w3_ref, b3_ref, out_ref)


BLK = 4096
NBLK = BK // BLK


def _mlp(emb, k, big, W1a, W1b, b1, W2, b2, W3t, b3):
    full = lambda shape: pl.BlockSpec(shape, lambda i: (0, 0))
    in_specs = [
        pl.BlockSpec((BLK, D), lambda i: (i, 0)),
        pl.BlockSpec((BLK, D), lambda i: (i + NBLK, 0)),
        full((D, D)),
        full((D, D)),
        full((1, D)),
        full((D, 64)),
        full((1, 64)),
        full((1, 64)),
        full((1, 1)),
    ]
    args = (emb, emb, W1a, W1b, b1, W2, b2, W3t, b3)
    body = _mlp_body
    kwargs = {}
    if big is not None:
        in_specs = [pl.BlockSpec(memory_space=pl.ANY)] + in_specs
        args = (big,) + args
        body = _mlp_alias_body
        kwargs = {"input_output_aliases": {0: 0}}
    return pl.pallas_call(
        body,
        grid=(NBLK,),
        in_specs=in_specs,
        out_specs=pl.BlockSpec((1, BLK), lambda i, k=k: (0, i + k * NBLK)),
        out_shape=jax.ShapeDtypeStruct((1, B), jnp.float32),
        **kwargs,
    )(*args)


def kernel(eng_ids, grk_ids, table, W1, b1, W2, b2, W3, b3):
    eng2d = eng_ids.astype(jnp.int32).reshape(B // CH, CH)
    grk2d = grk_ids.astype(jnp.int32).reshape(B // CH, CH)
    W1a, W1b = W1[:D], W1[D:]
    b1r = b1.reshape(1, D)
    b2r = b2.reshape(1, 64)
    W3t = W3.reshape(1, 64)
    b3r = b3.reshape(1, 1)
    out = None
    for k in range(K):
        emb = _gather_rows(table, eng2d, grk2d, k)
        out = _mlp(emb, k, out, W1a, W1b, b1r, W2, b2r, W3t, b3r)
    return out.reshape(B, 1)


# speedup vs baseline: 1.4046x; 1.0261x over previous
import functools

import jax
import jax.numpy as jnp
from jax import lax
from jax.experimental import pallas as pl
from jax.experimental.pallas import tpu as pltpu
from jax.experimental.pallas import tpu_sc as plsc

B = 16384
D = 128
NC, NS = 2, 16
NW = NC * NS
CH = 128
BLK = 4096

CHUNKS = (12288, 4096)


def _gather_rows(table, eng3, grk3, bk):
    rows = bk // CH // NW
    nchunk = 2 * rows
    mesh = plsc.VectorSubcoreMesh(
        core_axis_name="c", subcore_axis_name="s",
        num_cores=NC, num_subcores=NS)

    @functools.partial(
        pl.kernel,
        out_type=jax.ShapeDtypeStruct((2 * bk, D), jnp.float32),
        mesh=mesh,
        scratch_types=(
            [pltpu.VMEM((nchunk, CH), jnp.int32)]
            + [pltpu.VMEM((CH, D), jnp.float32)] * nchunk
            + [pltpu.SemaphoreType.DMA] * (2 * nchunk)
        ),
    )
    def gather_kernel(table_hbm, eng_hbm, grk_hbm, out_hbm, idx_v, *rest):
        bufs = rest[:nchunk]
        gsems = rest[nchunk:2 * nchunk]
        ssems = rest[2 * nchunk:]
        wid = lax.axis_index("s") * NC + lax.axis_index("c")
        pltpu.sync_copy(eng_hbm.at[wid], idx_v.at[pl.ds(0, rows)])
        pltpu.sync_copy(grk_hbm.at[wid], idx_v.at[pl.ds(rows, rows)])
        dst = [wid * rows * CH + j * CH for j in range(rows)]
        dst += [bk + wid * rows * CH + j * CH for j in range(rows)]
        gathers = [
            pltpu.async_copy(table_hbm.at[idx_v.at[j]], bufs[j], gsems[j])
            for j in range(nchunk)
        ]
        stores = [None] * nchunk
        for j in range(nchunk):
            gathers[j].wait()
            stores[j] = pltpu.async_copy(
                bufs[j], out_hbm.at[pl.ds(dst[j], CH)], ssems[j])
        for j in range(nchunk):
            stores[j].wait()

    return gather_kernel(table, eng3, grk3)


def _mlp_body(eng_ref, grk_ref, w1a_ref, w1b_ref, b1_ref, w2_ref, b2_ref,
              w3_ref, b3_ref, out_ref):
    h = eng_ref[...] @ w1a_ref[...] + grk_ref[...] @ w1b_ref[...] + b1_ref[...]
    h = jnp.maximum(h, 0.0)
    h = jnp.maximum(h @ w2_ref[...] + b2_ref[...], 0.0)
    zt = lax.dot_general(w3_ref[...], h, (((1,), (1,)), ((), ()))) + b3_ref[...]
    out_ref[...] = 1.0 / (1.0 + jnp.exp(-zt))


def _mlp_alias_body(big_ref, eng_ref, grk_ref, w1a_ref, w1b_ref, b1_ref,
                    w2_ref, b2_ref, w3_ref, b3_ref, out_ref):
    del big_ref
    _mlp_body(eng_ref, grk_ref, w1a_ref, w1b_ref, b1_ref, w2_ref, b2_ref,
              w3_ref, b3_ref, out_ref)


def _mlp(emb, bk, blk0, big, W1a, W1b, b1, W2, b2, W3t, b3):
    nblk = bk // BLK
    full = lambda shape: pl.BlockSpec(shape, lambda i: (0, 0))
    in_specs = [
        pl.BlockSpec((BLK, D), lambda i: (i, 0)),
        pl.BlockSpec((BLK, D), lambda i, nblk=nblk: (i + nblk, 0)),
        full((D, D)),
        full((D, D)),
        full((1, D)),
        full((D, 64)),
        full((1, 64)),
        full((1, 64)),
        full((1, 1)),
    ]
    args = (emb, emb, W1a, W1b, b1, W2, b2, W3t, b3)
    body = _mlp_body
    kwargs = {}
    if big is not None:
        in_specs = [pl.BlockSpec(memory_space=pl.ANY)] + in_specs
        args = (big,) + args
        body = _mlp_alias_body
        kwargs = {"input_output_aliases": {0: 0}}
    return pl.pallas_call(
        body,
        grid=(nblk,),
        in_specs=in_specs,
        out_specs=pl.BlockSpec((1, BLK), lambda i, blk0=blk0: (0, i + blk0)),
        out_shape=jax.ShapeDtypeStruct((1, B), jnp.float32),
        **kwargs,
    )(*args)


def kernel(eng_ids, grk_ids, table, W1, b1, W2, b2, W3, b3):
    eng2d = eng_ids.astype(jnp.int32).reshape(B // CH, CH)
    grk2d = grk_ids.astype(jnp.int32).reshape(B // CH, CH)
    W1a, W1b = W1[:D], W1[D:]
    b1r = b1.reshape(1, D)
    b2r = b2.reshape(1, 64)
    W3t = W3.reshape(1, 64)
    b3r = b3.reshape(1, 1)
    out = None
    row0 = 0
    for bk in CHUNKS:
        nrow = bk // CH
        e3 = lax.slice(eng2d, (row0, 0), (row0 + nrow, CH)).reshape(
            NW, nrow // NW, CH)
        g3 = lax.slice(grk2d, (row0, 0), (row0 + nrow, CH)).reshape(
            NW, nrow // NW, CH)
        emb = _gather_rows(table, e3, g3, bk)
        out = _mlp(emb, bk, row0 * CH // BLK, out,
                   W1a, W1b, b1r, W2, b2r, W3t, b3r)
        row0 += nrow
    return out.reshape(B, 1)
